# baseline (speedup 1.0000x reference)
import jax
import jax.numpy as jnp
from jax.experimental import pallas as pl


def _rowdot_kernel(gu_ref, gi_ref, out_ref):
    out_ref[...] = jnp.sum(gu_ref[...] * gi_ref[...], axis=1)


def kernel(inputs):
    gu = inputs[0]
    gi = inputs[1]
    n, d = gu.shape
    block = 2048
    return pl.pallas_call(
        _rowdot_kernel,
        grid=(n // block,),
        in_specs=[
            pl.BlockSpec((block, d), lambda i: (i, 0)),
            pl.BlockSpec((block, d), lambda i: (i, 0)),
        ],
        out_specs=pl.BlockSpec((block,), lambda i: (i,)),
        out_shape=jax.ShapeDtypeStruct((n,), gu.dtype),
    )(gu, gi)


# speedup vs baseline: 1.2813x; 1.2813x over previous
import jax
import jax.numpy as jnp
from jax.experimental import pallas as pl
from jax.experimental.pallas import tpu as pltpu


def _rowdot_kernel(gu_ref, gi_ref, out_ref):
    out_ref[...] = jnp.sum(gu_ref[0] * gi_ref[0], axis=1)


def kernel(inputs):
    n, d = inputs.shape[1], inputs.shape[2]
    block = 4096
    return pl.pallas_call(
        _rowdot_kernel,
        grid=(n // block,),
        in_specs=[
            pl.BlockSpec((1, block, d), lambda i: (0, i, 0)),
            pl.BlockSpec((1, block, d), lambda i: (1, i, 0)),
        ],
        out_specs=pl.BlockSpec((block,), lambda i: (i,)),
        out_shape=jax.ShapeDtypeStruct((n,), inputs.dtype),
        compiler_params=pltpu.CompilerParams(
            dimension_semantics=("arbitrary",),
        ),
    )(inputs, inputs)


# speedup vs baseline: 6.0762x; 4.7420x over previous
import jax
import jax.numpy as jnp
from jax.experimental import pallas as pl
from jax.experimental.pallas import tpu as pltpu


def _rowdot_kernel(gu_ref, gi_ref, out_ref):
    out_ref[...] = jnp.sum(gu_ref[0] * gi_ref[0], axis=0)


def kernel(inputs):
    n = inputs.shape[1]
    d = inputs.shape[2]
    t = jnp.swapaxes(inputs, 1, 2)
    block = 4096
    return pl.pallas_call(
        _rowdot_kernel,
        grid=(n // block,),
        in_specs=[
            pl.BlockSpec((1, d, block), lambda i: (0, 0, i)),
            pl.BlockSpec((1, d, block), lambda i: (1, 0, i)),
        ],
        out_specs=pl.BlockSpec((block,), lambda i: (i,)),
        out_shape=jax.ShapeDtypeStruct((n,), inputs.dtype),
        compiler_params=pltpu.CompilerParams(
            dimension_semantics=("arbitrary",),
        ),
    )(t, t)
